# Initial kernel scaffold; baseline (speedup 1.0000x reference)
import jax
import jax.numpy as jnp
from jax import lax
from jax.experimental import pallas as pl
from jax.experimental.pallas import tpu as pltpu
from jax.experimental.pallas import tpu_sc as plsc

L = 32768
K = 30
NC = 2
NS = 16
NW = NC * NS
ROWS_W = L // NW
CH = 128
CHUNKS = ROWS_W // CH
GRPS = CH // 16

_f32 = jnp.float32
_i32 = jnp.int32


def _sc_body(nc_hbm, dist_hbm, ang_hbm, seq_hbm,
             idx_hbm, dis_hbm, ango_hbm,
             seqv, ncf, distf, angf, idxo, diso, ango):
    c = lax.axis_index("c")
    s = lax.axis_index("s")
    wid = s * NC + c
    row0 = wid * ROWS_W
    pltpu.sync_copy(seq_hbm, seqv)
    lane = lax.iota(_i32, 16)
    zeros16 = jnp.zeros((16,), _f32)
    pad16 = jnp.full((16,), 22, _i32)
    ones16 = jnp.full((16,), 1, _i32)

    def chunk_body(ci, carry):
        r0 = row0 + ci * CH
        pltpu.sync_copy(nc_hbm.at[pl.ds(r0 * K, CH * K)], ncf)
        pltpu.sync_copy(dist_hbm.at[pl.ds(r0 * K, CH * K)], distf)
        pltpu.sync_copy(ang_hbm.at[pl.ds(r0 * K * 6, CH * K * 6)], angf)

        def ms_idx(i, cc):
            idxo[pl.ds(i * 16, 16)] = pad16
            return cc
        lax.fori_loop(0, CH * 22 // 16, ms_idx, 0)

        def ms_dis(i, cc):
            diso[pl.ds(i * 16, 16)] = zeros16
            return cc
        lax.fori_loop(0, CH * 22 // 16, ms_dis, 0)

        def ms_ang(i, cc):
            ango[pl.ds(i * 16, 16)] = zeros16
            return cc
        lax.fori_loop(0, CH * 132 // 16, ms_ang, 0)

        def grp_body(g, cc):
            lrow = g * 16 + lane
            rowv = r0 + lrow
            base_k = lrow * K
            base_a = lrow * (K * 6)
            base22 = lrow * 22
            base132 = lrow * 132
            cnt = jnp.zeros((16,), _i32)
            fb = jnp.zeros((16,), _i32)
            for k in range(K):
                ik = base_k + k
                v = plsc.load_gather(ncf, [ik])
                dk = plsc.load_gather(distf, [ik]) * _f32(0.1)
                sv = plsc.load_gather(seqv, [v])
                aks = [plsc.load_gather(angf, [base_a + (k * 6 + ch)])
                       * _f32(1.0 / 3.0) for ch in range(6)]
                d = v - rowv
                cond = jnp.abs(d) > 6
                w1 = cond & (cnt < 10)
                slot1 = jnp.minimum(cnt, 10)
                p1 = base22 + slot1
                plsc.store_scatter(idxo, [p1], sv, mask=w1)
                plsc.store_scatter(diso, [p1], dk, mask=w1)
                pa1 = base132 + slot1 * 6
                for ch in range(6):
                    plsc.store_scatter(ango, [pa1 + ch], aks[ch], mask=w1)
                cnt = cnt + cond.astype(_i32)
                inr = (d >= -6) & (d <= 6) & (d != 0)
                offi = jnp.where(inr, d + jnp.where(d < 0, 6, 5), 0)
                bit = jnp.left_shift(ones16, offi)
                newf = inr & ((fb & bit) == 0)
                fb = fb | jnp.where(inr, bit, 0)
                p2 = base22 + (10 + offi)
                plsc.store_scatter(idxo, [p2], sv, mask=newf)
                plsc.store_scatter(diso, [p2], dk, mask=newf)
                pa2 = base132 + (10 + offi) * 6
                for ch in range(6):
                    plsc.store_scatter(ango, [pa2 + ch], aks[ch], mask=newf)
            return cc
        lax.fori_loop(0, GRPS, grp_body, 0)

        pltpu.sync_copy(idxo, idx_hbm.at[pl.ds(r0 * 22, CH * 22)])
        pltpu.sync_copy(diso, dis_hbm.at[pl.ds(r0 * 22, CH * 22)])
        pltpu.sync_copy(ango, ango_hbm.at[pl.ds(r0 * 132, CH * 132)])
        return carry
    lax.fori_loop(0, CHUNKS, chunk_body, 0)


@jax.jit
def _sc_run(nc1, dist1, ang1, seq):
    mesh = plsc.VectorSubcoreMesh(core_axis_name="c", subcore_axis_name="s",
                                  num_cores=NC, num_subcores=NS)
    fn = pl.kernel(
        _sc_body,
        out_type=(
            jax.ShapeDtypeStruct((L * 22,), _i32),
            jax.ShapeDtypeStruct((L * 22,), _f32),
            jax.ShapeDtypeStruct((L * 132,), _f32),
        ),
        mesh=mesh,
        scratch_types=[
            pltpu.VMEM((L,), _i32),
            pltpu.VMEM((CH * K,), _i32),
            pltpu.VMEM((CH * K,), _f32),
            pltpu.VMEM((CH * K * 6,), _f32),
            pltpu.VMEM((CH * 22,), _i32),
            pltpu.VMEM((CH * 22,), _f32),
            pltpu.VMEM((CH * 132,), _f32),
        ],
    )
    return fn(nc1, dist1, ang1, seq)


def kernel(mask, num_cs, dist, angle, seqlist):
    Ln = mask.shape[0]
    nc1 = num_cs.astype(_i32).reshape(-1)
    dist1 = dist.reshape(-1)
    ang1 = angle.reshape(-1)
    seq = seqlist.astype(_i32)
    idx1d, dis_t, angle_t = _sc_run(nc1, dist1, ang1, seq)
    idx_t = idx1d.reshape(Ln, 22)
    data_t = jnp.eye(23, dtype=_f32)
    label = seqlist.astype(_i32)
    return (data_t, idx_t, dis_t, angle_t, label, Ln)


# speedup vs baseline: 12.3046x; 12.3046x over previous
import jax
import jax.numpy as jnp
from jax import lax
from jax.experimental import pallas as pl
from jax.experimental.pallas import tpu as pltpu
from jax.experimental.pallas import tpu_sc as plsc

L = 32768
K = 30
NC = 2
NS = 16
NW = NC * NS
ROWS_W = L // NW
CH = 128
CHUNKS = ROWS_W // CH
GRPS = CH // 16

_f32 = jnp.float32
_i32 = jnp.int32


def _sc_body(nc_hbm, dist_hbm, ang_hbm, seq_hbm,
             idx_hbm, dis_hbm, ango_hbm,
             seqv, ncf, distf, angf, idxo, diso, ango):
    c = lax.axis_index("c")
    s = lax.axis_index("s")
    wid = s * NC + c
    row0 = wid * ROWS_W
    pltpu.sync_copy(seq_hbm, seqv)
    lane = lax.iota(_i32, 16)
    zeros16 = jnp.zeros((16,), _f32)
    pad16 = jnp.full((16,), 22, _i32)
    ones16 = jnp.full((16,), 1, _i32)

    def chunk_body(ci, carry):
        r0 = row0 + ci * CH
        pltpu.sync_copy(nc_hbm.at[pl.ds(r0 * K, CH * K)], ncf)
        pltpu.sync_copy(dist_hbm.at[pl.ds(r0 * K, CH * K)], distf)
        pltpu.sync_copy(ang_hbm.at[pl.ds(r0 * K * 6, CH * K * 6)], angf)

        def ms_idx(i, cc):
            idxo[pl.ds(i * 16, 16)] = pad16
            return cc
        lax.fori_loop(0, CH * 22 // 16, ms_idx, 0)

        def ms_dis(i, cc):
            diso[pl.ds(i * 16, 16)] = zeros16
            return cc
        lax.fori_loop(0, CH * 22 // 16, ms_dis, 0)

        def ms_ang(i, cc):
            ango[pl.ds(i * 16, 16)] = zeros16
            return cc
        lax.fori_loop(0, CH * 132 // 16, ms_ang, 0)

        def grp_body(g, cc):
            lrow = g * 16 + lane
            rowv = r0 + lrow
            base_k = lrow * K
            base_a = lrow * (K * 6)
            base22 = lrow * 22
            base132 = lrow * 132
            cnt = jnp.zeros((16,), _i32)
            fb = jnp.zeros((16,), _i32)
            for k in range(K):
                ik = base_k + k
                v = plsc.load_gather(ncf, [ik])
                dk = plsc.load_gather(distf, [ik]) * _f32(0.1)
                sv = plsc.load_gather(seqv, [v])
                aks = [plsc.load_gather(angf, [base_a + (k * 6 + ch)])
                       * _f32(1.0 / 3.0) for ch in range(6)]
                d = v - rowv
                cond = jnp.abs(d) > 6
                w1 = cond & (cnt < 10)
                slot1 = jnp.minimum(cnt, 10)
                p1 = base22 + slot1
                plsc.store_scatter(idxo, [p1], sv, mask=w1)
                plsc.store_scatter(diso, [p1], dk, mask=w1)
                pa1 = base132 + slot1 * 6
                for ch in range(6):
                    plsc.store_scatter(ango, [pa1 + ch], aks[ch], mask=w1)
                cnt = cnt + cond.astype(_i32)
                inr = (d >= -6) & (d <= 6) & (d != 0)
                offi = jnp.where(inr, d + jnp.where(d < 0, 6, 5), 0)
                bit = jnp.left_shift(ones16, offi)
                newf = inr & ((fb & bit) == 0)
                fb = fb | jnp.where(inr, bit, 0)
                p2 = base22 + (10 + offi)
                plsc.store_scatter(idxo, [p2], sv, mask=newf)
                plsc.store_scatter(diso, [p2], dk, mask=newf)
                pa2 = base132 + (10 + offi) * 6
                for ch in range(6):
                    plsc.store_scatter(ango, [pa2 + ch], aks[ch], mask=newf)
            return cc
        lax.fori_loop(0, GRPS, grp_body, 0)

        pltpu.sync_copy(idxo, idx_hbm.at[pl.ds(r0 * 22, CH * 22)])
        pltpu.sync_copy(diso, dis_hbm.at[pl.ds(r0 * 22, CH * 22)])
        pltpu.sync_copy(ango, ango_hbm.at[pl.ds(r0 * 132, CH * 132)])
        return carry
    lax.fori_loop(0, CHUNKS, chunk_body, 0)


@jax.jit
def _sc_run(nc1, dist1, ang1, seq):
    mesh = plsc.VectorSubcoreMesh(core_axis_name="c", subcore_axis_name="s",
                                  num_cores=NC, num_subcores=NS)
    fn = pl.kernel(
        _sc_body,
        out_type=(
            jax.ShapeDtypeStruct((L * 22,), _i32),
            jax.ShapeDtypeStruct((L * 22,), _f32),
            jax.ShapeDtypeStruct((L * 132,), _f32),
        ),
        mesh=mesh,
        compiler_params=pltpu.CompilerParams(needs_layout_passes=False),
        scratch_types=[
            pltpu.VMEM((L,), _i32),
            pltpu.VMEM((CH * K,), _i32),
            pltpu.VMEM((CH * K,), _f32),
            pltpu.VMEM((CH * K * 6,), _f32),
            pltpu.VMEM((CH * 22,), _i32),
            pltpu.VMEM((CH * 22,), _f32),
            pltpu.VMEM((CH * 132,), _f32),
        ],
    )
    return fn(nc1, dist1, ang1, seq)


def kernel(mask, num_cs, dist, angle, seqlist):
    Ln = mask.shape[0]
    nc1 = num_cs.astype(_i32).reshape(-1)
    dist1 = dist.reshape(-1)
    ang1 = angle.reshape(-1)
    seq = seqlist.astype(_i32)
    idx1d, dis_t, angle_t = _sc_run(nc1, dist1, ang1, seq)
    idx_t = idx1d.reshape(Ln, 22)
    data_t = jnp.eye(23, dtype=_f32)
    label = seqlist.astype(_i32)
    return (data_t, idx_t, dis_t, angle_t, label, Ln)
